# baseline (speedup 1.0000x reference)
import functools

import jax
import jax.numpy as jnp
from jax import lax
from jax.experimental import pallas as pl
from jax.experimental.pallas import tpu as pltpu
from jax.experimental.pallas import tpu_sc as plsc

N = 10000
E = 160000
DIN = 512
D = 256
DH = 128
NC = 2
NS = 16
L = 16

RB = 1000

_mesh = lambda: plsc.VectorSubcoreMesh(core_axis_name="c", subcore_axis_name="s")


def _feat_body(x_ref, wt_ref, b_ref, hn_ref, hn2_ref, nrm_ref):
    h = jnp.dot(x_ref[...], wt_ref[...], preferred_element_type=jnp.float32)
    h = jnp.maximum(h + b_ref[...], 0.0)
    nrm = jnp.sqrt(jnp.sum(h * h, axis=1, keepdims=True))
    hn = h * (1.0 / jnp.maximum(nrm, 1e-12))
    hn_ref[...] = hn
    hn2_ref[0] = hn[:, :DH]
    hn2_ref[1] = hn[:, DH:]
    nrm_ref[...] = nrm


def _features(x, w1t, b1):
    return pl.pallas_call(
        _feat_body,
        grid=(N // RB,),
        in_specs=[
            pl.BlockSpec((RB, DIN), lambda i: (i, 0)),
            pl.BlockSpec((DIN, D), lambda i: (0, 0)),
            pl.BlockSpec((1, D), lambda i: (0, 0)),
        ],
        out_specs=[
            pl.BlockSpec((RB, D), lambda i: (i, 0)),
            pl.BlockSpec((2, RB, DH), lambda i: (0, i, 0)),
            pl.BlockSpec((RB, 1), lambda i: (i, 0)),
        ],
        out_shape=[
            jax.ShapeDtypeStruct((N, D), jnp.float32),
            jax.ShapeDtypeStruct((2, N, DH), jnp.float32),
            jax.ShapeDtypeStruct((N, 1), jnp.float32),
        ],
    )(x, w1t, b1)


C1 = 64
EPW = E // (NC * NS)
NCH1 = (EPW + C1 - 1) // C1
LAST1 = EPW - C1


def _edge_w_body(hn_hbm, src_hbm, dst_hbm, beta_hbm, w_hbm,
                 sidxa, didxa, srow0, drow0, srow1, drow1,
                 wbuf, bbuf, sem):
    c = lax.axis_index("c")
    s = lax.axis_index("s")
    wid = s * NC + c
    base0 = wid * EPW
    pltpu.sync_copy(beta_hbm, bbuf.at[pl.ds(0, 1)])
    pltpu.sync_copy(src_hbm.at[pl.ds(base0, EPW)], sidxa)
    pltpu.sync_copy(dst_hbm.at[pl.ds(base0, EPW)], didxa)
    b = bbuf[pl.ds(0, L)][0]
    lane = lax.iota(jnp.int32, L)

    def cbase(j):
        return jnp.minimum(j * C1, LAST1)

    def start(j, srow, drow):
        bs = cbase(j)
        pltpu.async_copy(hn_hbm.at[sidxa.at[pl.ds(bs, C1)]], srow, sem)
        pltpu.async_copy(hn_hbm.at[didxa.at[pl.ds(bs, C1)]], drow, sem)

    def drain(srow, drow):
        pltpu.make_async_copy(hn_hbm.at[pl.ds(0, C1)], srow, sem).wait()
        pltpu.make_async_copy(hn_hbm.at[pl.ds(0, C1)], drow, sem).wait()

    def compute(j, srow, drow):
        bs = cbase(j)

        def group(g, _):
            av = jnp.zeros((L,), jnp.float32)
            for e in range(L):
                r = g * L + e
                acc = jnp.zeros((L,), jnp.float32)
                for k in range(D // L):
                    acc = acc + (srow[r, pl.ds(k * L, L)]
                                 * drow[r, pl.ds(k * L, L)])
                av = jnp.where(lane == e, jnp.sum(acc), av)
            wbuf[pl.ds(g * L, L)] = jnp.exp(av * b)

        lax.fori_loop(0, C1 // L, group, None)
        pltpu.sync_copy(wbuf, w_hbm.at[pl.ds(base0 + bs, C1)])

    start(0, srow0, drow0)

    def pair(i, _):
        jb = 2 * i + 1

        @pl.when(jb < NCH1)
        def _():
            start(jb, srow1, drow1)
        drain(srow0, drow0)
        compute(2 * i, srow0, drow0)

        @pl.when(2 * i + 2 < NCH1)
        def _():
            start(2 * i + 2, srow0, drow0)

        @pl.when(jb < NCH1)
        def _():
            drain(srow1, drow1)
            compute(jb, srow1, drow1)

    lax.fori_loop(0, (NCH1 + 1) // 2, pair, None)


def _edge_w(hn, src, dst, beta):
    fn = functools.partial(
        pl.kernel,
        out_type=jax.ShapeDtypeStruct((E,), jnp.float32),
        mesh=_mesh(),
        compiler_params=pltpu.CompilerParams(needs_layout_passes=False),
        scratch_types=[
            pltpu.VMEM((EPW,), jnp.int32),
            pltpu.VMEM((EPW,), jnp.int32),
            pltpu.VMEM((C1, D), jnp.float32),
            pltpu.VMEM((C1, D), jnp.float32),
            pltpu.VMEM((C1, D), jnp.float32),
            pltpu.VMEM((C1, D), jnp.float32),
            pltpu.VMEM((C1,), jnp.float32),
            pltpu.VMEM((L,), jnp.float32),
            pltpu.SemaphoreType.DMA,
        ],
    )(_edge_w_body)
    return fn(hn, src, dst, beta)


TPR = 632
SCH = 2000
NPASS = 2
EPP = E // NPASS
CAP = 5440
C2 = 32


def _agg_body(hn2_hbm, src_hbm, dst_hbm, w_hbm, nrm_hbm, out_hbm,
              srcc, dstc, wc, srcm, dstm, wm, rows, nrmv, acc, accd, sem):
    c = lax.axis_index("c")
    s = lax.axis_index("s")
    cN = c * N
    lo = s * TPR
    npts = jnp.where(s == NS - 1, N - (NS - 1) * TPR, TPR)
    zv = jnp.zeros((L,), jnp.float32)

    def zacc(i, _):
        for k in range(DH // L):
            acc[i, pl.ds(k * L, L)] = zv
        accd[pl.ds(i * L, L)] = zv

    lax.fori_loop(0, TPR, zacc, None)
    pltpu.sync_copy(nrm_hbm, nrmv)

    for p in range(NPASS):
        cNv = jnp.full((L,), cN, jnp.int32)
        ziv = jnp.zeros((L,), jnp.int32)

        def pre(i, _):
            srcm[pl.ds(i * L, L)] = cNv
            dstm[pl.ds(i * L, L)] = ziv
            wm[pl.ds(i * L, L)] = zv

        lax.fori_loop(0, CAP // L, pre, None)

        def scan_chunk(i, cnt):
            base = p * EPP + i * SCH
            pltpu.sync_copy(src_hbm.at[pl.ds(base, SCH)], srcc)
            pltpu.sync_copy(dst_hbm.at[pl.ds(base, SCH)], dstc)
            pltpu.sync_copy(w_hbm.at[pl.ds(base, SCH)], wc)

            def grp(g, cnt):
                dv = dstc[pl.ds(g * L, L)]
                dl = dv - lo
                m = (dl >= 0) & (dl < npts)
                cc = jnp.minimum(cnt, CAP - L)
                plsc.store_compressed(srcm.at[pl.ds(cc, L)],
                                      srcc[pl.ds(g * L, L)] + cN, mask=m)
                plsc.store_compressed(dstm.at[pl.ds(cc, L)], dl, mask=m)
                plsc.store_compressed(wm.at[pl.ds(cc, L)],
                                      wc[pl.ds(g * L, L)], mask=m)
                return cnt + plsc.all_reduce_population_count(m)[0]

            return lax.fori_loop(0, SCH // L, grp, cnt)

        cnt = lax.fori_loop(0, EPP // SCH, scan_chunk, jnp.int32(0))

        def flush(j, _):
            @pl.when(j * C2 < cnt)
            def _():
                mb = j * C2
                pltpu.async_copy(
                    hn2_hbm.at[srcm.at[pl.ds(mb, C2)]], rows, sem).wait()

                def fgrp(g, _):
                    wvec = wm[pl.ds(mb + g * L, L)]
                    dlv = dstm[pl.ds(mb + g * L, L)]
                    iv = srcm[pl.ds(mb + g * L, L)] - cN
                    wpv = wvec * plsc.load_gather(nrmv, [iv])
                    for e in range(L):
                        r = g * L + e
                        wq = wpv[e]
                        dl = dlv[e]
                        for k in range(DH // L):
                            plsc.addupdate(acc.at[dl, pl.ds(k * L, L)],
                                           rows[r, pl.ds(k * L, L)] * wq)
                        plsc.addupdate(accd.at[pl.ds(dl * L, L)],
                                       jnp.full((L,), wvec[e], jnp.float32))

                lax.fori_loop(0, C2 // L, fgrp, None)

        lax.fori_loop(0, CAP // C2, flush, None)

    def divchunk(j, _):
        rb = jnp.minimum(j * C2, npts - C2)

        def rdiv(g, _):
            for e in range(L):
                r = rb + g * L + e
                den = accd[pl.ds(r * L, L)]
                rq = (1.0 / jnp.maximum(den, 1e-16))[0]
                for k in range(DH // L):
                    rows[g * L + e, pl.ds(k * L, L)] = (
                        acc[r, pl.ds(k * L, L)] * rq)

        lax.fori_loop(0, C2 // L, rdiv, None)
        pltpu.sync_copy(rows, out_hbm.at[c, pl.ds(lo + rb, C2)])

    lax.fori_loop(0, (TPR + C2 - 1) // C2, divchunk, None)


def _aggregate(hn2_flat, src, dst, w, nrm):
    fn = functools.partial(
        pl.kernel,
        out_type=jax.ShapeDtypeStruct((2, N, DH), jnp.float32),
        mesh=_mesh(),
        compiler_params=pltpu.CompilerParams(needs_layout_passes=False),
        scratch_types=[
            pltpu.VMEM((SCH,), jnp.int32),
            pltpu.VMEM((SCH,), jnp.int32),
            pltpu.VMEM((SCH,), jnp.float32),
            pltpu.VMEM((CAP,), jnp.int32),
            pltpu.VMEM((CAP,), jnp.int32),
            pltpu.VMEM((CAP,), jnp.float32),
            pltpu.VMEM((C2, DH), jnp.float32),
            pltpu.VMEM((N,), jnp.float32),
            pltpu.VMEM((TPR, DH), jnp.float32),
            pltpu.VMEM((TPR * L,), jnp.float32),
            pltpu.SemaphoreType.DMA,
        ],
    )(_agg_body)
    return fn(hn2_flat, src, dst, w, nrm)


def kernel(x, edge_index, W1, b1, beta):
    hn, hn2, nrm = _features(x, W1.T, b1.reshape(1, D))
    src = edge_index[0]
    dst = edge_index[1]
    w = _edge_w(hn, src, dst, beta)
    out2 = _aggregate(jnp.reshape(hn2, (2 * N, DH)), src, dst, w,
                      jnp.reshape(nrm, (N,)))
    return jnp.concatenate([out2[0], out2[1]], axis=1)


# speedup vs baseline: 1.4413x; 1.4413x over previous
import functools

import jax
import jax.numpy as jnp
from jax import lax
from jax.experimental import pallas as pl
from jax.experimental.pallas import tpu as pltpu
from jax.experimental.pallas import tpu_sc as plsc

N = 10000
E = 160000
DIN = 512
D = 256
DH = 128
NC = 2
NS = 16
L = 16

RB = 1000

_mesh = lambda: plsc.VectorSubcoreMesh(core_axis_name="c", subcore_axis_name="s")


def _feat_body(x_ref, wt_ref, b_ref, hn_ref, h2_ref):
    h = jnp.dot(x_ref[...], wt_ref[...], preferred_element_type=jnp.float32)
    h = jnp.maximum(h + b_ref[...], 0.0)
    nrm = jnp.sqrt(jnp.sum(h * h, axis=1, keepdims=True))
    hn = h * (1.0 / jnp.maximum(nrm, 1e-12))
    hn_ref[...] = hn
    h2_ref[0] = h[:, :DH]
    h2_ref[1] = h[:, DH:]


def _features(x, w1t, b1):
    return pl.pallas_call(
        _feat_body,
        grid=(N // RB,),
        in_specs=[
            pl.BlockSpec((RB, DIN), lambda i: (i, 0)),
            pl.BlockSpec((DIN, D), lambda i: (0, 0)),
            pl.BlockSpec((1, D), lambda i: (0, 0)),
        ],
        out_specs=[
            pl.BlockSpec((RB, D), lambda i: (i, 0)),
            pl.BlockSpec((2, RB, DH), lambda i: (0, i, 0)),
        ],
        out_shape=[
            jax.ShapeDtypeStruct((N, D), jnp.float32),
            jax.ShapeDtypeStruct((2, N, DH), jnp.float32),
        ],
    )(x, w1t, b1)


C1 = 64
EPW = E // (NC * NS)
NCH1 = (EPW + C1 - 1) // C1
LAST1 = EPW - C1


def _edge_w_body(hn_hbm, src_hbm, dst_hbm, beta_hbm, w_hbm,
                 sidxa, didxa, srow0, drow0, srow1, drow1,
                 wbuf, bbuf, sem):
    c = lax.axis_index("c")
    s = lax.axis_index("s")
    wid = s * NC + c
    base0 = wid * EPW
    pltpu.sync_copy(beta_hbm, bbuf.at[pl.ds(0, 1)])
    pltpu.sync_copy(src_hbm.at[pl.ds(base0, EPW)], sidxa)
    pltpu.sync_copy(dst_hbm.at[pl.ds(base0, EPW)], didxa)
    b = bbuf[pl.ds(0, L)][0]
    lane = lax.iota(jnp.int32, L)

    def cbase(j):
        return jnp.minimum(j * C1, LAST1)

    def start(j, srow, drow):
        bs = cbase(j)
        pltpu.async_copy(hn_hbm.at[sidxa.at[pl.ds(bs, C1)]], srow, sem)
        pltpu.async_copy(hn_hbm.at[didxa.at[pl.ds(bs, C1)]], drow, sem)

    def drain(srow, drow):
        pltpu.make_async_copy(hn_hbm.at[pl.ds(0, C1)], srow, sem).wait()
        pltpu.make_async_copy(hn_hbm.at[pl.ds(0, C1)], drow, sem).wait()

    def compute(j, srow, drow):
        bs = cbase(j)

        def group(g, _):
            av = jnp.zeros((L,), jnp.float32)
            for e in range(L):
                r = g * L + e
                acc = jnp.zeros((L,), jnp.float32)
                for k in range(D // L):
                    acc = acc + (srow[r, pl.ds(k * L, L)]
                                 * drow[r, pl.ds(k * L, L)])
                av = jnp.where(lane == e, jnp.sum(acc), av)
            wbuf[pl.ds(g * L, L)] = jnp.exp(av * b)

        lax.fori_loop(0, C1 // L, group, None)
        pltpu.sync_copy(wbuf, w_hbm.at[pl.ds(base0 + bs, C1)])

    start(0, srow0, drow0)

    def pair(i, _):
        jb = 2 * i + 1

        @pl.when(jb < NCH1)
        def _():
            start(jb, srow1, drow1)
        drain(srow0, drow0)
        compute(2 * i, srow0, drow0)

        @pl.when(2 * i + 2 < NCH1)
        def _():
            start(2 * i + 2, srow0, drow0)

        @pl.when(jb < NCH1)
        def _():
            drain(srow1, drow1)
            compute(jb, srow1, drow1)

    lax.fori_loop(0, (NCH1 + 1) // 2, pair, None)


def _edge_w(hn, src, dst, beta):
    fn = functools.partial(
        pl.kernel,
        out_type=jax.ShapeDtypeStruct((E,), jnp.float32),
        mesh=_mesh(),
        compiler_params=pltpu.CompilerParams(needs_layout_passes=False),
        scratch_types=[
            pltpu.VMEM((EPW,), jnp.int32),
            pltpu.VMEM((EPW,), jnp.int32),
            pltpu.VMEM((C1, D), jnp.float32),
            pltpu.VMEM((C1, D), jnp.float32),
            pltpu.VMEM((C1, D), jnp.float32),
            pltpu.VMEM((C1, D), jnp.float32),
            pltpu.VMEM((C1,), jnp.float32),
            pltpu.VMEM((L,), jnp.float32),
            pltpu.SemaphoreType.DMA,
        ],
    )(_edge_w_body)
    return fn(hn, src, dst, beta)


TPR = 632
SCH = 1600
NPASS = 2
EPP = E // NPASS
NSC = EPP // SCH
CAP = 5440
C2 = 32
NFL = CAP // C2


def _agg_body(h2_hbm, src_hbm, dst_hbm, w_hbm, out_hbm,
              srcc0, dstc0, wc0, srcc1, dstc1, wc1,
              srcm, dstm, wm, rows0, rows1, acc, accd, sems, semg):
    c = lax.axis_index("c")
    s = lax.axis_index("s")
    cN = c * N
    lo = s * TPR
    npts = jnp.where(s == NS - 1, N - (NS - 1) * TPR, TPR)
    zv = jnp.zeros((L,), jnp.float32)

    def zacc(i, _):
        for k in range(DH // L):
            acc[i, pl.ds(k * L, L)] = zv
        accd[pl.ds(i * L, L)] = zv

    lax.fori_loop(0, TPR, zacc, None)

    for p in range(NPASS):
        cNv = jnp.full((L,), cN, jnp.int32)
        ziv = jnp.zeros((L,), jnp.int32)

        def pre(i, _):
            srcm[pl.ds(i * L, L)] = cNv
            dstm[pl.ds(i * L, L)] = ziv
            wm[pl.ds(i * L, L)] = zv

        lax.fori_loop(0, CAP // L, pre, None)

        def sstart(i, sb, db, wb):
            base = p * EPP + i * SCH
            pltpu.async_copy(src_hbm.at[pl.ds(base, SCH)], sb, sems)
            pltpu.async_copy(dst_hbm.at[pl.ds(base, SCH)], db, sems)
            pltpu.async_copy(w_hbm.at[pl.ds(base, SCH)], wb, sems)

        def sdrain(sb, db, wb):
            pltpu.make_async_copy(src_hbm.at[pl.ds(0, SCH)], sb, sems).wait()
            pltpu.make_async_copy(dst_hbm.at[pl.ds(0, SCH)], db, sems).wait()
            pltpu.make_async_copy(w_hbm.at[pl.ds(0, SCH)], wb, sems).wait()

        def sproc(cnt, sb, db, wb):
            def grp(g, cnt):
                dv = db[pl.ds(g * L, L)]
                dl = dv - lo
                m = (dl >= 0) & (dl < npts)
                cc = jnp.minimum(cnt, CAP - L)
                plsc.store_compressed(srcm.at[pl.ds(cc, L)],
                                      sb[pl.ds(g * L, L)] + cN, mask=m)
                plsc.store_compressed(dstm.at[pl.ds(cc, L)], dl, mask=m)
                plsc.store_compressed(wm.at[pl.ds(cc, L)],
                                      wb[pl.ds(g * L, L)], mask=m)
                return cnt + plsc.all_reduce_population_count(m)[0]

            return lax.fori_loop(0, SCH // L, grp, cnt)

        sstart(0, srcc0, dstc0, wc0)

        def spair(i, cnt):
            sstart(2 * i + 1, srcc1, dstc1, wc1)
            sdrain(srcc0, dstc0, wc0)
            cnt = sproc(cnt, srcc0, dstc0, wc0)

            @pl.when(2 * i + 2 < NSC)
            def _():
                sstart(2 * i + 2, srcc0, dstc0, wc0)
            sdrain(srcc1, dstc1, wc1)
            return sproc(cnt, srcc1, dstc1, wc1)

        cnt = lax.fori_loop(0, NSC // 2, spair, jnp.int32(0))

        def fstart(j, rb):
            pltpu.async_copy(h2_hbm.at[srcm.at[pl.ds(j * C2, C2)]], rb, semg)

        def fproc(j, rb):
            pltpu.make_async_copy(h2_hbm.at[pl.ds(0, C2)], rb, semg).wait()
            mb = j * C2

            def fgrp(g, _):
                wvec = wm[pl.ds(mb + g * L, L)]
                dlv = dstm[pl.ds(mb + g * L, L)]
                for e in range(L):
                    r = g * L + e
                    wq = wvec[e]
                    dl = dlv[e]
                    for k in range(DH // L):
                        plsc.addupdate(acc.at[dl, pl.ds(k * L, L)],
                                       rb[r, pl.ds(k * L, L)] * wq)
                    plsc.addupdate(accd.at[pl.ds(dl * L, L)],
                                   jnp.full((L,), wq, jnp.float32))

            lax.fori_loop(0, C2 // L, fgrp, None)

        @pl.when(0 < cnt)
        def _():
            fstart(0, rows0)

        def fpair(i, _):
            ja = 2 * i
            jb = 2 * i + 1

            @pl.when(jb * C2 < cnt)
            def _():
                fstart(jb, rows1)

            @pl.when(ja * C2 < cnt)
            def _():
                fproc(ja, rows0)

            @pl.when((2 * i + 2) * C2 < cnt)
            def _():
                fstart(2 * i + 2, rows0)

            @pl.when(jb * C2 < cnt)
            def _():
                fproc(jb, rows1)

        lax.fori_loop(0, NFL // 2, fpair, None)

    def divchunk(j, _):
        rb = jnp.minimum(j * C2, npts - C2)

        def rdiv(g, _):
            for e in range(L):
                r = rb + g * L + e
                den = accd[pl.ds(r * L, L)]
                rq = (1.0 / jnp.maximum(den, 1e-16))[0]
                for k in range(DH // L):
                    rows0[g * L + e, pl.ds(k * L, L)] = (
                        acc[r, pl.ds(k * L, L)] * rq)

        lax.fori_loop(0, C2 // L, rdiv, None)
        pltpu.sync_copy(rows0, out_hbm.at[c, pl.ds(lo + rb, C2)])

    lax.fori_loop(0, (TPR + C2 - 1) // C2, divchunk, None)


def _aggregate(h2_flat, src, dst, w):
    fn = functools.partial(
        pl.kernel,
        out_type=jax.ShapeDtypeStruct((2, N, DH), jnp.float32),
        mesh=_mesh(),
        compiler_params=pltpu.CompilerParams(needs_layout_passes=False),
        scratch_types=[
            pltpu.VMEM((SCH,), jnp.int32),
            pltpu.VMEM((SCH,), jnp.int32),
            pltpu.VMEM((SCH,), jnp.float32),
            pltpu.VMEM((SCH,), jnp.int32),
            pltpu.VMEM((SCH,), jnp.int32),
            pltpu.VMEM((SCH,), jnp.float32),
            pltpu.VMEM((CAP,), jnp.int32),
            pltpu.VMEM((CAP,), jnp.int32),
            pltpu.VMEM((CAP,), jnp.float32),
            pltpu.VMEM((C2, DH), jnp.float32),
            pltpu.VMEM((C2, DH), jnp.float32),
            pltpu.VMEM((TPR, DH), jnp.float32),
            pltpu.VMEM((TPR * L,), jnp.float32),
            pltpu.SemaphoreType.DMA,
            pltpu.SemaphoreType.DMA,
        ],
    )(_agg_body)
    return fn(h2_flat, src, dst, w)


def kernel(x, edge_index, W1, b1, beta):
    hn, h2 = _features(x, W1.T, b1.reshape(1, D))
    src = edge_index[0]
    dst = edge_index[1]
    w = _edge_w(hn, src, dst, beta)
    out2 = _aggregate(jnp.reshape(h2, (2 * N, DH)), src, dst, w)
    return jnp.concatenate([out2[0], out2[1]], axis=1)


# speedup vs baseline: 2.5466x; 1.7669x over previous
import functools

import jax
import jax.numpy as jnp
from jax import lax
from jax.experimental import pallas as pl
from jax.experimental.pallas import tpu as pltpu
from jax.experimental.pallas import tpu_sc as plsc

N = 10000
E = 160000
DIN = 512
D = 256
DH = 128
NC = 2
NS = 16
L = 16

RB = 1000

_mesh = lambda: plsc.VectorSubcoreMesh(core_axis_name="c", subcore_axis_name="s")


def _feat_body(x_ref, wt_ref, b_ref, hn_ref, h2_ref):
    h = jnp.dot(x_ref[...], wt_ref[...], preferred_element_type=jnp.float32)
    h = jnp.maximum(h + b_ref[...], 0.0)
    nrm = jnp.sqrt(jnp.sum(h * h, axis=1, keepdims=True))
    hn = h * (1.0 / jnp.maximum(nrm, 1e-12))
    hn_ref[...] = hn
    h2_ref[0] = h[:, :DH]
    h2_ref[1] = h[:, DH:]


def _features(x, w1t, b1):
    return pl.pallas_call(
        _feat_body,
        grid=(N // RB,),
        in_specs=[
            pl.BlockSpec((RB, DIN), lambda i: (i, 0)),
            pl.BlockSpec((DIN, D), lambda i: (0, 0)),
            pl.BlockSpec((1, D), lambda i: (0, 0)),
        ],
        out_specs=[
            pl.BlockSpec((RB, D), lambda i: (i, 0)),
            pl.BlockSpec((2, RB, DH), lambda i: (0, i, 0)),
        ],
        out_shape=[
            jax.ShapeDtypeStruct((N, D), jnp.float32),
            jax.ShapeDtypeStruct((2, N, DH), jnp.float32),
        ],
    )(x, w1t, b1)


C1 = 64
EPW = E // (NC * NS)
NCH1 = (EPW + C1 - 1) // C1
LAST1 = EPW - C1


def _edge_w_body(hn_hbm, src_hbm, dst_hbm, beta_hbm, w_hbm,
                 sidxa, didxa, srow0, drow0, srow1, drow1,
                 wbuf, bbuf, sem):
    c = lax.axis_index("c")
    s = lax.axis_index("s")
    wid = s * NC + c
    base0 = wid * EPW
    pltpu.sync_copy(beta_hbm, bbuf.at[pl.ds(0, 1)])
    pltpu.sync_copy(src_hbm.at[pl.ds(base0, EPW)], sidxa)
    pltpu.sync_copy(dst_hbm.at[pl.ds(base0, EPW)], didxa)
    b = bbuf[pl.ds(0, L)][0]
    lane = lax.iota(jnp.int32, L)

    def cbase(j):
        return jnp.minimum(j * C1, LAST1)

    def start(j, srow, drow):
        bs = cbase(j)
        pltpu.async_copy(hn_hbm.at[sidxa.at[pl.ds(bs, C1)]], srow, sem)
        pltpu.async_copy(hn_hbm.at[didxa.at[pl.ds(bs, C1)]], drow, sem)

    def drain(srow, drow):
        pltpu.make_async_copy(hn_hbm.at[pl.ds(0, C1)], srow, sem).wait()
        pltpu.make_async_copy(hn_hbm.at[pl.ds(0, C1)], drow, sem).wait()

    def compute(j, srow, drow):
        bs = cbase(j)

        def group(g, _):
            av = jnp.zeros((L,), jnp.float32)
            for e in range(L):
                r = g * L + e
                acc = jnp.zeros((L,), jnp.float32)
                for k in range(D // L):
                    acc = acc + (srow[r, pl.ds(k * L, L)]
                                 * drow[r, pl.ds(k * L, L)])
                av = jnp.where(lane == e, jnp.sum(acc), av)
            wbuf[pl.ds(g * L, L)] = jnp.exp(av * b)

        lax.fori_loop(0, C1 // L, group, None)
        pltpu.sync_copy(wbuf, w_hbm.at[pl.ds(base0 + bs, C1)])

    start(0, srow0, drow0)

    def pair(i, _):
        jb = 2 * i + 1

        @pl.when(jb < NCH1)
        def _():
            start(jb, srow1, drow1)
        drain(srow0, drow0)
        compute(2 * i, srow0, drow0)

        @pl.when(2 * i + 2 < NCH1)
        def _():
            start(2 * i + 2, srow0, drow0)

        @pl.when(jb < NCH1)
        def _():
            drain(srow1, drow1)
            compute(jb, srow1, drow1)

    lax.fori_loop(0, (NCH1 + 1) // 2, pair, None)


def _edge_w(hn, src, dst, beta):
    fn = functools.partial(
        pl.kernel,
        out_type=jax.ShapeDtypeStruct((E,), jnp.float32),
        mesh=_mesh(),
        compiler_params=pltpu.CompilerParams(needs_layout_passes=False),
        scratch_types=[
            pltpu.VMEM((EPW,), jnp.int32),
            pltpu.VMEM((EPW,), jnp.int32),
            pltpu.VMEM((C1, D), jnp.float32),
            pltpu.VMEM((C1, D), jnp.float32),
            pltpu.VMEM((C1, D), jnp.float32),
            pltpu.VMEM((C1, D), jnp.float32),
            pltpu.VMEM((C1,), jnp.float32),
            pltpu.VMEM((L,), jnp.float32),
            pltpu.SemaphoreType.DMA,
        ],
    )(_edge_w_body)
    return fn(hn, src, dst, beta)


EPT = E // NS
C2 = 32
NCH2 = EPT // C2
CT = EPT - NCH2 * C2
DVC = 640


def _agg_body(h2_hbm, src_hbm, dst_hbm, w_hbm, out_hbm,
              sidxa, didxa, wva, didxc0, didxc1,
              rows0, rows1, wrow0, wrow1, zbuf1, denc,
              acc_sp, den_sp, semg, sems):
    c = lax.axis_index("c")
    s = lax.axis_index("s")
    cN = c * N
    e0 = s * EPT
    zv = jnp.zeros((L,), jnp.float32)

    pltpu.sync_copy(src_hbm.at[pl.ds(e0, EPT)], sidxa)
    pltpu.sync_copy(dst_hbm.at[pl.ds(e0, EPT)], didxa)
    pltpu.sync_copy(w_hbm.at[pl.ds(e0, EPT)], wva)

    def addcn(i, _):
        sidxa[pl.ds(i * L, L)] = sidxa[pl.ds(i * L, L)] + cN

    lax.fori_loop(0, EPT // L, addcn, None)

    def zw(i, _):
        for k in range(DH // L):
            wrow0[i, pl.ds(k * L, L)] = zv

    lax.fori_loop(0, C2, zw, None)

    def z1(i, _):
        zbuf1[pl.ds(i * L, L)] = zv

    lax.fori_loop(0, 2000 // L, z1, None)

    for j in range(625 // C2):
        pltpu.sync_copy(wrow0, acc_sp.at[pl.ds(s * 625 + j * C2, C2)])
    pltpu.sync_copy(wrow0.at[pl.ds(0, 625 - (625 // C2) * C2)],
                    acc_sp.at[pl.ds(s * 625 + (625 // C2) * C2,
                                    625 - (625 // C2) * C2)])

    @pl.when(s == 0)
    def _():
        for j in range(5):
            pltpu.sync_copy(zbuf1, den_sp.at[pl.ds(j * 2000, 2000)])

    plsc.subcore_barrier()

    def gstart(j, rows):
        pltpu.async_copy(h2_hbm.at[sidxa.at[pl.ds(j * C2, C2)]], rows, semg)

    def gdrain(rows):
        pltpu.make_async_copy(h2_hbm.at[pl.ds(0, C2)], rows, semg).wait()

    def sdrain(j, wrow, didxc):
        pltpu.make_async_copy(wrow, acc_sp.at[didxc], sems).wait()
        pltpu.make_async_copy(wva.at[pl.ds(j * C2, C2)],
                              den_sp.at[didxc], sems).wait()

    def compute(j, rows, wrow, didxc):
        bs = j * C2
        gdrain(rows)
        for g in range(C2 // L):
            didxc[pl.ds(g * L, L)] = didxa[pl.ds(bs + g * L, L)]

        def fgrp(g, _):
            wvec = wva[pl.ds(bs + g * L, L)]
            for e in range(L):
                r = g * L + e
                wq = wvec[e]
                for k in range(DH // L):
                    wrow[r, pl.ds(k * L, L)] = rows[r, pl.ds(k * L, L)] * wq

        lax.fori_loop(0, C2 // L, fgrp, None)
        pltpu.async_copy(wrow, acc_sp.at[didxc], sems, add=True)
        pltpu.async_copy(wva.at[pl.ds(bs, C2)], den_sp.at[didxc],
                         sems, add=True)

    gstart(0, rows0)

    def pair(i, _):
        ja = 2 * i
        jb = 2 * i + 1
        gstart(jb, rows1)

        @pl.when(i >= 1)
        def _():
            sdrain(ja - 2, wrow0, didxc0)
        compute(ja, rows0, wrow0, didxc0)

        @pl.when(ja + 2 < NCH2)
        def _():
            gstart(ja + 2, rows0)

        @pl.when(i >= 1)
        def _():
            sdrain(jb - 2, wrow1, didxc1)
        compute(jb, rows1, wrow1, didxc1)

    lax.fori_loop(0, NCH2 // 2, pair, None)
    sdrain(NCH2 - 2, wrow0, didxc0)
    sdrain(NCH2 - 1, wrow1, didxc1)

    bs = NCH2 * C2
    pltpu.async_copy(h2_hbm.at[sidxa.at[pl.ds(bs, CT)]],
                     rows0.at[pl.ds(0, CT)], semg)
    pltpu.make_async_copy(h2_hbm.at[pl.ds(0, CT)],
                          rows0.at[pl.ds(0, CT)], semg).wait()
    for g in range(CT // L):
        didxc0[pl.ds(g * L, L)] = didxa[pl.ds(bs + g * L, L)]

    def tgrp(g, _):
        wvec = wva[pl.ds(bs + g * L, L)]
        for e in range(L):
            r = g * L + e
            wq = wvec[e]
            for k in range(DH // L):
                wrow0[r, pl.ds(k * L, L)] = rows0[r, pl.ds(k * L, L)] * wq

    lax.fori_loop(0, CT // L, tgrp, None)
    pltpu.sync_copy(wrow0.at[pl.ds(0, CT)],
                    acc_sp.at[didxc0.at[pl.ds(0, CT)]], add=True)
    pltpu.sync_copy(wva.at[pl.ds(bs, CT)],
                    den_sp.at[didxc0.at[pl.ds(0, CT)]], add=True)

    plsc.subcore_barrier()

    def divchunk(j, _):
        @pl.when((s < NS - 1) | (j < 12))
        def _():
            row0 = s * DVC + j * C2
            pltpu.sync_copy(acc_sp.at[pl.ds(row0, C2)], rows0)
            pltpu.sync_copy(den_sp.at[pl.ds(row0, C2)], denc)

            def rdiv(g, _):
                rv = 1.0 / jnp.maximum(denc[pl.ds(g * L, L)], 1e-16)
                for e in range(L):
                    r = g * L + e
                    rq = rv[e]
                    for k in range(DH // L):
                        wrow0[r, pl.ds(k * L, L)] = rows0[r, pl.ds(k * L, L)] * rq

            lax.fori_loop(0, C2 // L, rdiv, None)
            pltpu.sync_copy(wrow0, out_hbm.at[c, pl.ds(row0, C2)])

    lax.fori_loop(0, DVC // C2, divchunk, None)

    @pl.when(s == NS - 1)
    def _():
        row0 = N - L
        pltpu.sync_copy(acc_sp.at[pl.ds(row0, L)], rows0.at[pl.ds(0, L)])
        pltpu.sync_copy(den_sp.at[pl.ds(row0, L)], denc.at[pl.ds(0, L)])
        rv = 1.0 / jnp.maximum(denc[pl.ds(0, L)], 1e-16)
        for e in range(L):
            rq = rv[e]
            for k in range(DH // L):
                wrow0[e, pl.ds(k * L, L)] = rows0[e, pl.ds(k * L, L)] * rq
        pltpu.sync_copy(wrow0.at[pl.ds(0, L)], out_hbm.at[c, pl.ds(row0, L)])


def _aggregate(h2_flat, src, dst, w):
    fn = functools.partial(
        pl.kernel,
        out_type=jax.ShapeDtypeStruct((2, N, DH), jnp.float32),
        mesh=_mesh(),
        compiler_params=pltpu.CompilerParams(needs_layout_passes=False),
        scratch_types=[
            pltpu.VMEM((EPT,), jnp.int32),
            pltpu.VMEM((EPT,), jnp.int32),
            pltpu.VMEM((EPT,), jnp.float32),
            pltpu.VMEM((C2,), jnp.int32),
            pltpu.VMEM((C2,), jnp.int32),
            pltpu.VMEM((C2, DH), jnp.float32),
            pltpu.VMEM((C2, DH), jnp.float32),
            pltpu.VMEM((C2, DH), jnp.float32),
            pltpu.VMEM((C2, DH), jnp.float32),
            pltpu.VMEM((2000,), jnp.float32),
            pltpu.VMEM((C2,), jnp.float32),
            pltpu.VMEM_SHARED((N, DH), jnp.float32),
            pltpu.VMEM_SHARED((N,), jnp.float32),
            pltpu.SemaphoreType.DMA,
            pltpu.SemaphoreType.DMA,
        ],
    )(_agg_body)
    return fn(h2_flat, src, dst, w)


def kernel(x, edge_index, W1, b1, beta):
    hn, h2 = _features(x, W1.T, b1.reshape(1, D))
    src = edge_index[0]
    dst = edge_index[1]
    w = _edge_w(hn, src, dst, beta)
    out2 = _aggregate(jnp.reshape(h2, (2 * N, DH)), src, dst, w)
    return jnp.concatenate([out2[0], out2[1]], axis=1)


# speedup vs baseline: 3.0042x; 1.1797x over previous
import functools

import jax
import jax.numpy as jnp
from jax import lax
from jax.experimental import pallas as pl
from jax.experimental.pallas import tpu as pltpu
from jax.experimental.pallas import tpu_sc as plsc

N = 10000
E = 160000
DIN = 512
D = 256
DH = 128
NC = 2
NS = 16
L = 16

RB = 1000

_mesh = lambda: plsc.VectorSubcoreMesh(core_axis_name="c", subcore_axis_name="s")


def _feat_body(x_ref, wt_ref, b_ref, hnb_ref, h2_ref):
    h = jnp.dot(x_ref[...], wt_ref[...], preferred_element_type=jnp.float32)
    h = jnp.maximum(h + b_ref[...], 0.0)
    nrm = jnp.sqrt(jnp.sum(h * h, axis=1, keepdims=True))
    hn = h * (1.0 / jnp.maximum(nrm, 1e-12))
    hnb_ref[...] = hn.astype(jnp.bfloat16)
    h2_ref[0] = h[:, :DH]
    h2_ref[1] = h[:, DH:]


def _features(x, w1t, b1):
    return pl.pallas_call(
        _feat_body,
        grid=(N // RB,),
        in_specs=[
            pl.BlockSpec((RB, DIN), lambda i: (i, 0)),
            pl.BlockSpec((DIN, D), lambda i: (0, 0)),
            pl.BlockSpec((1, D), lambda i: (0, 0)),
        ],
        out_specs=[
            pl.BlockSpec((RB, D), lambda i: (i, 0)),
            pl.BlockSpec((2, RB, DH), lambda i: (0, i, 0)),
        ],
        out_shape=[
            jax.ShapeDtypeStruct((N, D), jnp.bfloat16),
            jax.ShapeDtypeStruct((2, N, DH), jnp.float32),
        ],
    )(x, w1t, b1)


C1 = 64
EPW = E // (NC * NS)
NCH1 = (EPW + C1 - 1) // C1
LAST1 = EPW - C1


def _edge_w_body(hn_hbm, src_hbm, dst_hbm, beta_hbm, w_hbm,
                 sidxa, didxa, srow0, drow0, srow1, drow1,
                 wbuf, bbuf, sem):
    c = lax.axis_index("c")
    s = lax.axis_index("s")
    wid = s * NC + c
    base0 = wid * EPW
    pltpu.sync_copy(beta_hbm, bbuf.at[pl.ds(0, 1)])
    pltpu.sync_copy(src_hbm.at[pl.ds(base0, EPW)], sidxa)
    pltpu.sync_copy(dst_hbm.at[pl.ds(base0, EPW)], didxa)
    b = bbuf[pl.ds(0, L)][0]
    lane = lax.iota(jnp.int32, L)

    def cbase(j):
        return jnp.minimum(j * C1, LAST1)

    def start(j, srow, drow):
        bs = cbase(j)
        pltpu.async_copy(hn_hbm.at[sidxa.at[pl.ds(bs, C1)]], srow, sem)
        pltpu.async_copy(hn_hbm.at[didxa.at[pl.ds(bs, C1)]], drow, sem)

    def drain(srow, drow):
        pltpu.make_async_copy(hn_hbm.at[pl.ds(0, C1)], srow, sem).wait()
        pltpu.make_async_copy(hn_hbm.at[pl.ds(0, C1)], drow, sem).wait()

    def compute(j, srow, drow):
        bs = cbase(j)

        def group(g, _):
            av = jnp.zeros((L,), jnp.float32)
            for e in range(L):
                r = g * L + e
                acc = jnp.zeros((L,), jnp.float32)
                for k in range(D // (2 * L)):
                    sv = plsc.bitcast(srow[r, pl.ds(k * L, L)], jnp.bfloat16)
                    dv = plsc.bitcast(drow[r, pl.ds(k * L, L)], jnp.bfloat16)
                    sa, sb = plsc.unpack(
                        sv, format=plsc.PackFormat.INTERLEAVED,
                        preferred_element_type=jnp.float32)
                    da, db = plsc.unpack(
                        dv, format=plsc.PackFormat.INTERLEAVED,
                        preferred_element_type=jnp.float32)
                    acc = acc + sa * da + sb * db
                av = jnp.where(lane == e, jnp.sum(acc), av)
            wbuf[pl.ds(g * L, L)] = jnp.exp(av * b)

        lax.fori_loop(0, C1 // L, group, None)
        pltpu.sync_copy(wbuf, w_hbm.at[pl.ds(base0 + bs, C1)])

    start(0, srow0, drow0)

    def pair(i, _):
        jb = 2 * i + 1

        @pl.when(jb < NCH1)
        def _():
            start(jb, srow1, drow1)
        drain(srow0, drow0)
        compute(2 * i, srow0, drow0)

        @pl.when(2 * i + 2 < NCH1)
        def _():
            start(2 * i + 2, srow0, drow0)

        @pl.when(jb < NCH1)
        def _():
            drain(srow1, drow1)
            compute(jb, srow1, drow1)

    lax.fori_loop(0, (NCH1 + 1) // 2, pair, None)


def _edge_w(hn, src, dst, beta):
    fn = functools.partial(
        pl.kernel,
        out_type=jax.ShapeDtypeStruct((E,), jnp.float32),
        mesh=_mesh(),
        compiler_params=pltpu.CompilerParams(needs_layout_passes=False),
        scratch_types=[
            pltpu.VMEM((EPW,), jnp.int32),
            pltpu.VMEM((EPW,), jnp.int32),
            pltpu.VMEM((C1, DH), jnp.int32),
            pltpu.VMEM((C1, DH), jnp.int32),
            pltpu.VMEM((C1, DH), jnp.int32),
            pltpu.VMEM((C1, DH), jnp.int32),
            pltpu.VMEM((C1,), jnp.float32),
            pltpu.VMEM((L,), jnp.float32),
            pltpu.SemaphoreType.DMA,
        ],
    )(_edge_w_body)
    return fn(hn, src, dst, beta)


EPT = E // NS
C2 = 32
NCH2 = EPT // C2
CT = EPT - NCH2 * C2
DVC = 640


def _agg_body(h2_hbm, src_hbm, dst_hbm, w_hbm, out_hbm,
              sidxa, didxa, wva, didxc0, didxc1,
              rows0, rows1, wrow0, wrow1, zbuf1, denc,
              acc_sp, den_sp, semg, sems):
    c = lax.axis_index("c")
    s = lax.axis_index("s")
    cN = c * N
    e0 = s * EPT
    zv = jnp.zeros((L,), jnp.float32)

    pltpu.sync_copy(src_hbm.at[pl.ds(e0, EPT)], sidxa)
    pltpu.sync_copy(dst_hbm.at[pl.ds(e0, EPT)], didxa)
    pltpu.sync_copy(w_hbm.at[pl.ds(e0, EPT)], wva)

    def addcn(i, _):
        sidxa[pl.ds(i * L, L)] = sidxa[pl.ds(i * L, L)] + cN

    lax.fori_loop(0, EPT // L, addcn, None)

    def zw(i, _):
        for k in range(DH // L):
            wrow0[i, pl.ds(k * L, L)] = zv

    lax.fori_loop(0, C2, zw, None)

    def z1(i, _):
        zbuf1[pl.ds(i * L, L)] = zv

    lax.fori_loop(0, 2000 // L, z1, None)

    for j in range(625 // C2):
        pltpu.sync_copy(wrow0, acc_sp.at[pl.ds(s * 625 + j * C2, C2)])
    pltpu.sync_copy(wrow0.at[pl.ds(0, 625 - (625 // C2) * C2)],
                    acc_sp.at[pl.ds(s * 625 + (625 // C2) * C2,
                                    625 - (625 // C2) * C2)])

    @pl.when(s == 0)
    def _():
        for j in range(5):
            pltpu.sync_copy(zbuf1, den_sp.at[pl.ds(j * 2000, 2000)])

    plsc.subcore_barrier()

    def gstart(j, rows):
        pltpu.async_copy(h2_hbm.at[sidxa.at[pl.ds(j * C2, C2)]], rows, semg)

    def gdrain(rows):
        pltpu.make_async_copy(h2_hbm.at[pl.ds(0, C2)], rows, semg).wait()

    def sdrain(j, wrow, didxc):
        pltpu.make_async_copy(wrow, acc_sp.at[didxc], sems).wait()
        pltpu.make_async_copy(wva.at[pl.ds(j * C2, C2)],
                              den_sp.at[didxc], sems).wait()

    def compute(j, rows, wrow, didxc):
        bs = j * C2
        gdrain(rows)
        for g in range(C2 // L):
            didxc[pl.ds(g * L, L)] = didxa[pl.ds(bs + g * L, L)]

        def fgrp(g, _):
            wvec = wva[pl.ds(bs + g * L, L)]
            for e in range(L):
                r = g * L + e
                wq = wvec[e]
                for k in range(DH // L):
                    wrow[r, pl.ds(k * L, L)] = rows[r, pl.ds(k * L, L)] * wq

        lax.fori_loop(0, C2 // L, fgrp, None)
        pltpu.async_copy(wrow, acc_sp.at[didxc], sems, add=True)
        pltpu.async_copy(wva.at[pl.ds(bs, C2)], den_sp.at[didxc],
                         sems, add=True)

    gstart(0, rows0)

    def pair(i, _):
        ja = 2 * i
        jb = 2 * i + 1
        gstart(jb, rows1)

        @pl.when(i >= 1)
        def _():
            sdrain(ja - 2, wrow0, didxc0)
        compute(ja, rows0, wrow0, didxc0)

        @pl.when(ja + 2 < NCH2)
        def _():
            gstart(ja + 2, rows0)

        @pl.when(i >= 1)
        def _():
            sdrain(jb - 2, wrow1, didxc1)
        compute(jb, rows1, wrow1, didxc1)

    lax.fori_loop(0, NCH2 // 2, pair, None)
    sdrain(NCH2 - 2, wrow0, didxc0)
    sdrain(NCH2 - 1, wrow1, didxc1)

    bs = NCH2 * C2
    pltpu.async_copy(h2_hbm.at[sidxa.at[pl.ds(bs, CT)]],
                     rows0.at[pl.ds(0, CT)], semg)
    pltpu.make_async_copy(h2_hbm.at[pl.ds(0, CT)],
                          rows0.at[pl.ds(0, CT)], semg).wait()
    for g in range(CT // L):
        didxc0[pl.ds(g * L, L)] = didxa[pl.ds(bs + g * L, L)]

    def tgrp(g, _):
        wvec = wva[pl.ds(bs + g * L, L)]
        for e in range(L):
            r = g * L + e
            wq = wvec[e]
            for k in range(DH // L):
                wrow0[r, pl.ds(k * L, L)] = rows0[r, pl.ds(k * L, L)] * wq

    lax.fori_loop(0, CT // L, tgrp, None)
    pltpu.sync_copy(wrow0.at[pl.ds(0, CT)],
                    acc_sp.at[didxc0.at[pl.ds(0, CT)]], add=True)
    pltpu.sync_copy(wva.at[pl.ds(bs, CT)],
                    den_sp.at[didxc0.at[pl.ds(0, CT)]], add=True)

    plsc.subcore_barrier()

    def divchunk(j, _):
        @pl.when((s < NS - 1) | (j < 12))
        def _():
            row0 = s * DVC + j * C2
            pltpu.sync_copy(acc_sp.at[pl.ds(row0, C2)], rows0)
            pltpu.sync_copy(den_sp.at[pl.ds(row0, C2)], denc)

            def rdiv(g, _):
                rv = 1.0 / jnp.maximum(denc[pl.ds(g * L, L)], 1e-16)
                for e in range(L):
                    r = g * L + e
                    rq = rv[e]
                    for k in range(DH // L):
                        wrow0[r, pl.ds(k * L, L)] = rows0[r, pl.ds(k * L, L)] * rq

            lax.fori_loop(0, C2 // L, rdiv, None)
            pltpu.sync_copy(wrow0, out_hbm.at[c, pl.ds(row0, C2)])

    lax.fori_loop(0, DVC // C2, divchunk, None)

    @pl.when(s == NS - 1)
    def _():
        row0 = N - L
        pltpu.sync_copy(acc_sp.at[pl.ds(row0, L)], rows0.at[pl.ds(0, L)])
        pltpu.sync_copy(den_sp.at[pl.ds(row0, L)], denc.at[pl.ds(0, L)])
        rv = 1.0 / jnp.maximum(denc[pl.ds(0, L)], 1e-16)
        for e in range(L):
            rq = rv[e]
            for k in range(DH // L):
                wrow0[e, pl.ds(k * L, L)] = rows0[e, pl.ds(k * L, L)] * rq
        pltpu.sync_copy(wrow0.at[pl.ds(0, L)], out_hbm.at[c, pl.ds(row0, L)])


def _aggregate(h2_flat, src, dst, w):
    fn = functools.partial(
        pl.kernel,
        out_type=jax.ShapeDtypeStruct((2, N, DH), jnp.float32),
        mesh=_mesh(),
        compiler_params=pltpu.CompilerParams(needs_layout_passes=False),
        scratch_types=[
            pltpu.VMEM((EPT,), jnp.int32),
            pltpu.VMEM((EPT,), jnp.int32),
            pltpu.VMEM((EPT,), jnp.float32),
            pltpu.VMEM((C2,), jnp.int32),
            pltpu.VMEM((C2,), jnp.int32),
            pltpu.VMEM((C2, DH), jnp.float32),
            pltpu.VMEM((C2, DH), jnp.float32),
            pltpu.VMEM((C2, DH), jnp.float32),
            pltpu.VMEM((C2, DH), jnp.float32),
            pltpu.VMEM((2000,), jnp.float32),
            pltpu.VMEM((C2,), jnp.float32),
            pltpu.VMEM_SHARED((N, DH), jnp.float32),
            pltpu.VMEM_SHARED((N,), jnp.float32),
            pltpu.SemaphoreType.DMA,
            pltpu.SemaphoreType.DMA,
        ],
    )(_agg_body)
    return fn(h2_flat, src, dst, w)


def kernel(x, edge_index, W1, b1, beta):
    hnb, h2 = _features(x, W1.T, b1.reshape(1, D))
    hnb32 = lax.bitcast_convert_type(hnb.reshape(N, DH, 2), jnp.int32)
    src = edge_index[0]
    dst = edge_index[1]
    w = _edge_w(hnb32, src, dst, beta)
    out2 = _aggregate(jnp.reshape(h2, (2 * N, DH)), src, dst, w)
    return jnp.concatenate([out2[0], out2[1]], axis=1)
